# baseline (speedup 1.0000x reference)
import functools

import jax
import jax.numpy as jnp
from jax import lax
from jax.experimental import pallas as pl
from jax.experimental.pallas import tpu as pltpu
from jax.experimental.pallas import tpu_sc as plsc

DIM = 128
C = 5
LANES = 16
DCH = DIM // LANES

NC = 2
NS = 16


def _make_sc_kernel(batch):
    nw = NC * NS
    b_per_w = batch // nw
    chunk = 64
    nch = b_per_w // chunk
    npair = nch // 2

    mesh = plsc.VectorSubcoreMesh(
        core_axis_name="c", subcore_axis_name="s",
        num_cores=NC, num_subcores=NS)

    @functools.partial(
        pl.kernel,
        out_type=jax.ShapeDtypeStruct((batch * C,), jnp.float32),
        mesh=mesh,
        scratch_types=[
            pltpu.VMEM((b_per_w,), jnp.int32),
            pltpu.VMEM((b_per_w * C,), jnp.int32),
            pltpu.VMEM((2, chunk, DIM), jnp.float32),
            pltpu.VMEM((2, chunk * C, DIM), jnp.float32),
            pltpu.VMEM((chunk * C + LANES,), jnp.float32),
            pltpu.SemaphoreType.DMA,
            pltpu.SemaphoreType.DMA,
        ],
    )
    def sc_kernel(tgt_hbm, ctx_hbm, tt_hbm, ct_hbm, out_hbm,
                  idx_t, idx_c, we_v, ce_v, out_v, sem0, sem1):
        wid = lax.axis_index("s") * NC + lax.axis_index("c")
        base = wid * b_per_w
        pltpu.sync_copy(tgt_hbm.at[pl.ds(base, b_per_w)], idx_t)
        pltpu.sync_copy(ctx_hbm.at[pl.ds(base * C, b_per_w * C)], idx_c)
        sems = [sem0, sem1]

        def descs(ch, buf, make):
            cps = [make(
                tt_hbm.at[idx_t.at[pl.ds(ch * chunk, chunk)]],
                we_v.at[buf], sems[buf])]
            coff = ch * chunk * C
            for g in range(C):
                cps.append(make(
                    ct_hbm.at[idx_c.at[pl.ds(coff + g * chunk, chunk)]],
                    ce_v.at[buf, pl.ds(g * chunk, chunk)], sems[buf]))
            return cps

        def fire(ch, buf):
            descs(ch, buf, pltpu.async_copy)

        def wait_chunk(ch, buf):
            for cp in descs(ch, buf, pltpu.make_async_copy):
                cp.wait()

        lane = lax.iota(jnp.int32, LANES)
        perms = [lane ^ (1 << t) for t in range(3, -1, -1)]
        onehot = [jnp.where(lane == m, 1.0, 0.0).astype(jnp.float32)
                  for m in range(C)]

        def compute(buf, ch):
            def body(i2, _):
                rows = [i2 * 2, i2 * 2 + 1]
                we = [[we_v[buf, i, pl.ds(k * LANES, LANES)]
                       for k in range(DCH)] for i in rows]
                nd = 2 * C
                acc = [None] * nd
                for k in range(DCH):
                    for d in range(nd):
                        r, c = divmod(d, C)
                        t = ce_v[buf, rows[r] * C + c,
                                 pl.ds(k * LANES, LANES)] * we[r][k]
                        acc[d] = t if acc[d] is None else acc[d] + t
                for p in perms:
                    shf = [acc[d].at[p].get(mode="promise_in_bounds")
                           for d in range(nd)]
                    acc = [acc[d] + shf[d] for d in range(nd)]
                for r in range(2):
                    comb = acc[r * C] * onehot[0]
                    for c in range(1, C):
                        comb += acc[r * C + c] * onehot[c]
                    out_v[pl.ds(rows[r] * C, LANES)] = comb
                return 0

            lax.fori_loop(0, chunk // 2, body, 0)
            pltpu.sync_copy(
                out_v.at[pl.ds(0, chunk * C)],
                out_hbm.at[pl.ds((base + ch * chunk) * C, chunk * C)])

        fire(0, 0)

        def pair(g, _):
            ch0 = g * 2
            wait_chunk(ch0, 0)
            fire(ch0 + 1, 1)
            compute(0, ch0)
            wait_chunk(ch0 + 1, 1)

            @pl.when(g + 1 < npair)
            def _():
                fire(ch0 + 2, 0)

            compute(1, ch0 + 1)
            return 0

        lax.fori_loop(0, npair, pair, 0)

    return sc_kernel


def kernel(target, context, target_table, context_table):
    batch = target.shape[0]
    half = batch // 4
    sck = _make_sc_kernel(half)
    outs = []
    for s in range(4):
        tgt_flat = lax.slice_in_dim(target, s * half, (s + 1) * half,
                                    axis=0).reshape(half)
        ctx_flat = lax.slice_in_dim(context, s * half, (s + 1) * half,
                                    axis=0).reshape(half * C)
        out_flat = sck(tgt_flat, ctx_flat, target_table, context_table)
        outs.append(out_flat.reshape(half, C))
    return jnp.concatenate(outs, axis=0)


# speedup vs baseline: 1.1884x; 1.1884x over previous
import functools

import jax
import jax.numpy as jnp
from jax import lax
from jax.experimental import pallas as pl
from jax.experimental.pallas import tpu as pltpu
from jax.experimental.pallas import tpu_sc as plsc

DIM = 128
C = 5
LANES = 16
DCH = DIM // LANES

NC = 2
NS = 16


def _make_sc_kernel(batch):
    nw = NC * NS
    b_per_w = batch // nw
    chunk = 64
    nch = b_per_w // chunk
    npair = nch // 2

    mesh = plsc.VectorSubcoreMesh(
        core_axis_name="c", subcore_axis_name="s",
        num_cores=NC, num_subcores=NS)

    @functools.partial(
        pl.kernel,
        out_type=jax.ShapeDtypeStruct((batch * C,), jnp.float32),
        mesh=mesh,
        scratch_types=[
            pltpu.VMEM((b_per_w,), jnp.int32),
            pltpu.VMEM((b_per_w * C,), jnp.int32),
            pltpu.VMEM((2, chunk, DIM), jnp.float32),
            pltpu.VMEM((2, chunk * C, DIM), jnp.float32),
            pltpu.VMEM((chunk * C + LANES,), jnp.float32),
            pltpu.SemaphoreType.DMA,
            pltpu.SemaphoreType.DMA,
        ],
    )
    def sc_kernel(tgt_hbm, ctx_hbm, tt_hbm, ct_hbm, out_hbm,
                  idx_t, idx_c, we_v, ce_v, out_v, sem0, sem1):
        wid = lax.axis_index("s") * NC + lax.axis_index("c")
        base = wid * b_per_w
        pltpu.sync_copy(tgt_hbm.at[pl.ds(base, b_per_w)], idx_t)
        pltpu.sync_copy(ctx_hbm.at[pl.ds(base * C, b_per_w * C)], idx_c)
        sems = [sem0, sem1]

        def descs(ch, buf, make):
            cps = [make(
                tt_hbm.at[idx_t.at[pl.ds(ch * chunk, chunk)]],
                we_v.at[buf], sems[buf])]
            coff = ch * chunk * C
            for g in range(C):
                cps.append(make(
                    ct_hbm.at[idx_c.at[pl.ds(coff + g * chunk, chunk)]],
                    ce_v.at[buf, pl.ds(g * chunk, chunk)], sems[buf]))
            return cps

        def fire(ch, buf):
            descs(ch, buf, pltpu.async_copy)

        def wait_chunk(ch, buf):
            for cp in descs(ch, buf, pltpu.make_async_copy):
                cp.wait()

        lane = lax.iota(jnp.int32, LANES)
        perms = [lane ^ (1 << t) for t in range(3, -1, -1)]
        onehot = [jnp.where(lane == m, 1.0, 0.0).astype(jnp.float32)
                  for m in range(C)]

        def compute(buf, ch):
            def body(i2, _):
                rows = [i2 * 2, i2 * 2 + 1]
                we = [[we_v[buf, i, pl.ds(k * LANES, LANES)]
                       for k in range(DCH)] for i in rows]
                nd = 2 * C
                acc = [None] * nd
                for k in range(DCH):
                    for d in range(nd):
                        r, c = divmod(d, C)
                        t = ce_v[buf, rows[r] * C + c,
                                 pl.ds(k * LANES, LANES)] * we[r][k]
                        acc[d] = t if acc[d] is None else acc[d] + t
                for p in perms:
                    shf = [acc[d].at[p].get(mode="promise_in_bounds")
                           for d in range(nd)]
                    acc = [acc[d] + shf[d] for d in range(nd)]
                for r in range(2):
                    comb = acc[r * C] * onehot[0]
                    for c in range(1, C):
                        comb += acc[r * C + c] * onehot[c]
                    out_v[pl.ds(rows[r] * C, LANES)] = comb
                return 0

            lax.fori_loop(0, chunk // 2, body, 0)
            pltpu.sync_copy(
                out_v.at[pl.ds(0, chunk * C)],
                out_hbm.at[pl.ds((base + ch * chunk) * C, chunk * C)])

        fire(0, 0)

        def pair(g, _):
            ch0 = g * 2
            wait_chunk(ch0, 0)
            fire(ch0 + 1, 1)
            compute(0, ch0)
            wait_chunk(ch0 + 1, 1)

            @pl.when(g + 1 < npair)
            def _():
                fire(ch0 + 2, 0)

            compute(1, ch0 + 1)
            return 0

        lax.fori_loop(0, npair, pair, 0)

    return sc_kernel


def kernel(target, context, target_table, context_table):
    batch = target.shape[0]
    half = batch // 2
    sck = _make_sc_kernel(half)
    outs = []
    for s in range(2):
        tgt_flat = lax.slice_in_dim(target, s * half, (s + 1) * half,
                                    axis=0).reshape(half)
        ctx_flat = lax.slice_in_dim(context, s * half, (s + 1) * half,
                                    axis=0).reshape(half * C)
        out_flat = sck(tgt_flat, ctx_flat, target_table, context_table)
        outs.append(out_flat.reshape(half, C))
    return jnp.concatenate(outs, axis=0)
